# baseline (speedup 1.0000x reference)
import jax
import jax.numpy as jnp
from jax import lax
from jax.experimental import pallas as pl
from jax.experimental.pallas import tpu as pltpu, tpu_sc as plsc

_NC, _NS = 2, 16
_NW = _NC * _NS
_R = 16


def _tc_body(x_ref, pos_ref, o_ref):
    o_ref[...] = x_ref[...] + pos_ref[...][None, :, :]


def _tc_kernel(x, pos_table, bt=256):
    b, t, d = x.shape
    if t % bt != 0:
        bt = t
    return pl.pallas_call(
        _tc_body,
        grid=(t // bt,),
        in_specs=[
            pl.BlockSpec((b, bt, d), lambda i: (0, i, 0)),
            pl.BlockSpec((bt, d), lambda i: (i, 0)),
        ],
        out_specs=pl.BlockSpec((b, bt, d), lambda i: (0, i, 0)),
        out_shape=jax.ShapeDtypeStruct((b, t, d), x.dtype),
    )(x, pos_table[:t])


def _sc_kernel(x, pos_table):
    b, t, d = x.shape
    n = _R * d
    rows_w = t // _NW
    chunks = rows_w // _R

    def body(x_hbm, pos_hbm, out_hbm, pos_v, xw_v):
        wid = lax.axis_index("s") * _NC + lax.axis_index("c")
        base = wid * rows_w
        for c in range(chunks):
            r0 = base + c * _R
            pltpu.sync_copy(pos_hbm.at[pl.ds(r0 * d, n)], pos_v)
            for bb in range(b):
                off = (bb * t + r0) * d
                pltpu.sync_copy(x_hbm.at[pl.ds(off, n)], xw_v)

                @plsc.parallel_loop(0, n, step=16, unroll=8)
                def _(o):
                    plsc.addupdate(xw_v.at[pl.ds(o, 16)], pos_v[pl.ds(o, 16)])

                pltpu.sync_copy(xw_v, out_hbm.at[pl.ds(off, n)])

    out = pl.kernel(
        body,
        out_type=jax.ShapeDtypeStruct((b * t * d,), x.dtype),
        mesh=plsc.VectorSubcoreMesh(core_axis_name="c", subcore_axis_name="s"),
        scratch_types=[
            pltpu.VMEM((n,), jnp.float32),
            pltpu.VMEM((n,), jnp.float32),
        ],
    )(x.reshape(b * t * d), pos_table[:t].reshape(t * d))
    return out.reshape(b, t, d)


def kernel(x, pos_table):
    return _sc_kernel(x, pos_table)


# speedup vs baseline: 1.2325x; 1.2325x over previous
import jax
import jax.numpy as jnp
from jax import lax
from jax.experimental import pallas as pl
from jax.experimental.pallas import tpu as pltpu, tpu_sc as plsc

_NC, _NS = 2, 16
_NW = _NC * _NS
_R = 8
_LEAD = 2


def _sc_kernel(x, pos_table):
    b, t, d = x.shape
    n = _R * d
    rows_w = t // _NW
    chunks = rows_w // _R
    upg = 2 * b
    groups = (chunks * b) // upg

    def body(x_hbm, pos_hbm, out_hbm, pos0, pos1, xw0, xw1, xw2, xw3,
             psem0, psem1, isem0, isem1, isem2, isem3,
             osem0, osem1, osem2, osem3):
        pos_v = (pos0, pos1)
        xw = (xw0, xw1, xw2, xw3)
        psem = (psem0, psem1)
        isem = (isem0, isem1, isem2, isem3)
        osem = (osem0, osem1, osem2, osem3)

        wid = lax.axis_index("s") * _NC + lax.axis_index("c")
        base = wid * rows_w

        def pos_slice(c):
            return pos_hbm.at[pl.ds((base + c * _R) * d, n)]

        def x_slice(c, bb):
            return x_hbm.at[pl.ds(((bb * t) + base + c * _R) * d, n)]

        def out_slice(c, bb):
            return out_hbm.at[pl.ds(((bb * t) + base + c * _R) * d, n)]

        pltpu.async_copy(pos_slice(0), pos_v[0], psem[0])
        for u in range(_LEAD):
            pltpu.async_copy(x_slice(0, u), xw[u], isem[u])

        @pl.loop(0, groups)
        def _(g):
            for uu in range(upg):
                cc, bb = divmod(uu, b)
                k = uu % b
                c = 2 * g + cc

                if uu == 0:
                    pltpu.async_copy(pos_slice(c + 1), pos_v[1], psem[1])
                    pltpu.make_async_copy(pos_slice(c), pos_v[0],
                                          psem[0]).wait()
                if uu == b:
                    @pl.when(g < groups - 1)
                    def _():
                        pltpu.async_copy(pos_slice(c + 1), pos_v[0], psem[0])
                    pltpu.make_async_copy(pos_slice(c), pos_v[1],
                                          psem[1]).wait()

                pltpu.make_async_copy(x_slice(c, bb), xw[k], isem[k]).wait()

                @plsc.parallel_loop(0, n, step=16, unroll=8)
                def _(o):
                    plsc.addupdate(xw[k].at[pl.ds(o, 16)],
                                   pos_v[cc][pl.ds(o, 16)])

                pltpu.async_copy(xw[k], out_slice(c, bb), osem[k])

                vcc, vbb = divmod(uu + _LEAD, b)
                vk = (uu + _LEAD) % b
                vc = 2 * g + vcc

                def _service(vc=vc, vbb=vbb, vk=vk, c=c, bb=bb):
                    def _drain():
                        pltpu.make_async_copy(xw[vk], out_slice(c, bb),
                                              osem[vk]).wait()

                    if uu < _LEAD:
                        pl.when(g > 0)(_drain)
                    else:
                        _drain()
                    pltpu.async_copy(x_slice(vc, vbb), xw[vk], isem[vk])

                if uu + _LEAD < upg:
                    _service()
                else:
                    pl.when(g < groups - 1)(_service)

        for u in range(chunks * b - 2 * _LEAD, chunks * b):
            k = u % b
            c, bb = divmod(u, b)
            pltpu.make_async_copy(xw[k], out_slice(c, bb), osem[k]).wait()

    out = pl.kernel(
        body,
        out_type=jax.ShapeDtypeStruct((b * t * d,), x.dtype),
        mesh=plsc.VectorSubcoreMesh(core_axis_name="c", subcore_axis_name="s"),
        scratch_types=[
            pltpu.VMEM((n,), jnp.float32),
            pltpu.VMEM((n,), jnp.float32),
            pltpu.VMEM((n,), jnp.float32),
            pltpu.VMEM((n,), jnp.float32),
            pltpu.VMEM((n,), jnp.float32),
            pltpu.VMEM((n,), jnp.float32),
            pltpu.SemaphoreType.DMA,
            pltpu.SemaphoreType.DMA,
            pltpu.SemaphoreType.DMA,
            pltpu.SemaphoreType.DMA,
            pltpu.SemaphoreType.DMA,
            pltpu.SemaphoreType.DMA,
            pltpu.SemaphoreType.DMA,
            pltpu.SemaphoreType.DMA,
            pltpu.SemaphoreType.DMA,
            pltpu.SemaphoreType.DMA,
        ],
    )(x.reshape(b * t * d), pos_table[:t].reshape(t * d))
    return out.reshape(b, t, d)


def kernel(x, pos_table):
    return _sc_kernel(x, pos_table)


# speedup vs baseline: 3.5263x; 2.8610x over previous
import jax
import jax.numpy as jnp
from jax import lax
from jax.experimental import pallas as pl
from jax.experimental.pallas import tpu as pltpu, tpu_sc as plsc

_NC, _NS = 2, 16
_NW = _NC * _NS
_R = 8
_LEAD = 2


def _sc_kernel(x, pos_table):
    b, t, d = x.shape
    rows_w = t // _NW
    chunks = rows_w // _R
    upg = 2 * b
    groups = (chunks * b) // upg

    def body(x_hbm, pos_hbm, out_hbm, pos0, pos1, xw0, xw1, xw2, xw3,
             pidx0, pidx1, xidx0, xidx1, xidx2, xidx3,
             psem0, psem1, isem0, isem1, isem2, isem3,
             osem0, osem1, osem2, osem3):
        pos_v = (pos0, pos1)
        xw = (xw0, xw1, xw2, xw3)
        pidx = (pidx0, pidx1)
        xidx = (xidx0, xidx1, xidx2, xidx3)
        psem = (psem0, psem1)
        isem = (isem0, isem1, isem2, isem3)
        osem = (osem0, osem1, osem2, osem3)

        wid = lax.axis_index("s") * _NC + lax.axis_index("c")
        base = wid * rows_w
        iota = lax.iota(jnp.int32, 16)

        def pos_rows(c):
            return base + c * _R

        def x_rows(c, bb):
            return bb * t + base + c * _R

        def stage_pos(c, p):
            pidx[p][...] = pos_rows(c) + iota
            pltpu.async_copy(pos_hbm.at[pidx[p].at[pl.ds(0, _R)]],
                             pos_v[p], psem[p])

        def wait_pos(p):
            pltpu.make_async_copy(pos_hbm.at[pidx[p].at[pl.ds(0, _R)]],
                                  pos_v[p], psem[p]).wait()

        def stage_x(c, bb, k):
            xidx[k][...] = x_rows(c, bb) + iota
            pltpu.async_copy(x_hbm.at[xidx[k].at[pl.ds(0, _R)]],
                             xw[k], isem[k])

        def wait_x(k):
            pltpu.make_async_copy(x_hbm.at[xidx[k].at[pl.ds(0, _R)]],
                                  xw[k], isem[k]).wait()

        def store_out(k):
            pltpu.async_copy(xw[k], out_hbm.at[xidx[k].at[pl.ds(0, _R)]],
                             osem[k])

        def drain_out(k):
            pltpu.make_async_copy(xw[k], out_hbm.at[xidx[k].at[pl.ds(0, _R)]],
                                  osem[k]).wait()

        stage_pos(0, 0)
        for u in range(_LEAD):
            stage_x(0, u, u)

        @pl.loop(0, groups)
        def _(g):
            for uu in range(upg):
                cc, bb = divmod(uu, b)
                k = uu % b
                c = 2 * g + cc

                if uu == 0:
                    stage_pos(c + 1, 1)
                    wait_pos(0)
                if uu == b:
                    @pl.when(g < groups - 1)
                    def _():
                        stage_pos(c + 1, 0)
                    wait_pos(1)

                wait_x(k)

                for r in range(_R):
                    @plsc.parallel_loop(0, d, step=16, unroll=8)
                    def _(o):
                        plsc.addupdate(xw[k].at[r, pl.ds(o, 16)],
                                       pos_v[cc][r, pl.ds(o, 16)])

                store_out(k)

                vcc, vbb = divmod(uu + _LEAD, b)
                vk = (uu + _LEAD) % b
                vc = 2 * g + vcc

                def _service(vc=vc, vbb=vbb, vk=vk):
                    drain_out(vk)
                    stage_x(vc, vbb, vk)

                if uu + _LEAD < upg:
                    if uu < _LEAD:
                        pl.when(g > 0)(lambda vk=vk: drain_out(vk))
                        stage_x(vc, vbb, vk)
                    else:
                        _service()
                else:
                    pl.when(g < groups - 1)(_service)

        for u in range(chunks * b - 2 * _LEAD, chunks * b):
            k = u % b
            drain_out(k)

    out = pl.kernel(
        body,
        out_type=jax.ShapeDtypeStruct((b * t, d), x.dtype),
        mesh=plsc.VectorSubcoreMesh(core_axis_name="c", subcore_axis_name="s"),
        scratch_types=[
            pltpu.VMEM((_R, d), jnp.float32),
            pltpu.VMEM((_R, d), jnp.float32),
            pltpu.VMEM((_R, d), jnp.float32),
            pltpu.VMEM((_R, d), jnp.float32),
            pltpu.VMEM((_R, d), jnp.float32),
            pltpu.VMEM((_R, d), jnp.float32),
            pltpu.VMEM((16,), jnp.int32),
            pltpu.VMEM((16,), jnp.int32),
            pltpu.VMEM((16,), jnp.int32),
            pltpu.VMEM((16,), jnp.int32),
            pltpu.VMEM((16,), jnp.int32),
            pltpu.VMEM((16,), jnp.int32),
            pltpu.SemaphoreType.DMA,
            pltpu.SemaphoreType.DMA,
            pltpu.SemaphoreType.DMA,
            pltpu.SemaphoreType.DMA,
            pltpu.SemaphoreType.DMA,
            pltpu.SemaphoreType.DMA,
            pltpu.SemaphoreType.DMA,
            pltpu.SemaphoreType.DMA,
            pltpu.SemaphoreType.DMA,
            pltpu.SemaphoreType.DMA,
        ],
    )(x.reshape(b * t, d), pos_table[:t])
    return out.reshape(b, t, d)


def kernel(x, pos_table):
    return _sc_kernel(x, pos_table)
